# baseline (speedup 1.0000x reference)
import functools

import jax
import jax.numpy as jnp
from jax import lax
from jax.experimental import pallas as pl
from jax.experimental.pallas import tpu as pltpu
from jax.experimental.pallas import tpu_sc as plsc

VOCAB = 100000
D_EMB = 128
N_PROMPT = 20
BATCH = 4
SEQ_LEN = 2048

_ROWS = BATCH * SEQ_LEN
_NW = 32
_R_PER_W = _ROWS // _NW
_CHUNK = 64
_NCHUNK = _R_PER_W // _CHUNK
_W_PER_BATCH = _NW // BATCH


def _make_kernel():
    mesh = plsc.VectorSubcoreMesh(core_axis_name="c", subcore_axis_name="s")

    @functools.partial(
        pl.kernel,
        mesh=mesh,
        out_type=jax.ShapeDtypeStruct((_ROWS, D_EMB), jnp.float32),
        scratch_types=[
            pltpu.VMEM((_NCHUNK, _CHUNK), jnp.int32),
            pltpu.VMEM((_R_PER_W, D_EMB), jnp.float32),
        ]
        + [pltpu.SemaphoreType.DMA] * _NCHUNK,
    )
    def k(x_hbm, table_hbm, prompt_hbm, out_hbm, idx_v, rows_v, *sems):
        nc = 2
        wid = lax.axis_index("s") * nc + lax.axis_index("c")
        base = wid * _R_PER_W
        pltpu.sync_copy(x_hbm.at[pl.ds(_NCHUNK * wid, _NCHUNK)], idx_v)
        gathers = [
            pltpu.async_copy(
                table_hbm.at[idx_v.at[j]],
                rows_v.at[pl.ds(j * _CHUNK, _CHUNK)],
                sems[j],
            )
            for j in range(_NCHUNK)
        ]
        writes = []
        for j in range(_NCHUNK):
            gathers[j].wait()
            if j == 0:
                @pl.when(wid % _W_PER_BATCH == 0)
                def _():
                    pltpu.sync_copy(prompt_hbm, rows_v.at[pl.ds(0, N_PROMPT)])

            writes.append(
                pltpu.async_copy(
                    rows_v.at[pl.ds(j * _CHUNK, _CHUNK)],
                    out_hbm.at[pl.ds(base + j * _CHUNK, _CHUNK)],
                    sems[j],
                )
            )
        for w in writes:
            w.wait()

    return k


_kernel_call = _make_kernel()


def kernel(x, table, prompt):
    x2 = x.reshape(_ROWS // _CHUNK, _CHUNK)
    out = _kernel_call(x2, table, prompt.reshape(N_PROMPT, D_EMB))
    return out.reshape(BATCH, SEQ_LEN, D_EMB)


# speedup vs baseline: 1.0417x; 1.0417x over previous
import functools

import jax
import jax.numpy as jnp
from jax import lax
from jax.experimental import pallas as pl
from jax.experimental.pallas import tpu as pltpu
from jax.experimental.pallas import tpu_sc as plsc

VOCAB = 100000
D_EMB = 128
N_PROMPT = 20
BATCH = 4
SEQ_LEN = 2048

_ROWS = BATCH * SEQ_LEN
_NW = 32
_R_PER_W = _ROWS // _NW
_CHUNK = 64
_NCHUNK = _R_PER_W // _CHUNK
_W_PER_BATCH = _NW // BATCH


def _make_kernel():
    mesh = plsc.VectorSubcoreMesh(core_axis_name="c", subcore_axis_name="s")

    @functools.partial(
        pl.kernel,
        mesh=mesh,
        out_type=jax.ShapeDtypeStruct((_ROWS, D_EMB), jnp.float32),
        scratch_types=[
            pltpu.VMEM((_NCHUNK, _CHUNK), jnp.int32),
            pltpu.VMEM((_R_PER_W, D_EMB), jnp.float32),
            pltpu.VMEM((24, D_EMB), jnp.float32),
            pltpu.SemaphoreType.DMA,
        ]
        + [pltpu.SemaphoreType.DMA] * _NCHUNK,
    )
    def k(x_hbm, table_hbm, prompt_hbm, out_hbm, idx_v, rows_v, prompt_v,
          psem, *sems):
        wid = lax.axis_index("c") * 16 + lax.axis_index("s")
        base = wid * _R_PER_W
        is_head = wid % _W_PER_BATCH == 0
        prompt_cp = pltpu.async_copy(prompt_hbm, prompt_v.at[pl.ds(0, N_PROMPT)], psem)
        pltpu.sync_copy(x_hbm.at[pl.ds(_NCHUNK * wid, _NCHUNK)], idx_v)
        gathers = [
            pltpu.async_copy(
                table_hbm.at[idx_v.at[j]],
                rows_v.at[pl.ds(j * _CHUNK, _CHUNK)],
                sems[j],
            )
            for j in range(_NCHUNK)
        ]
        writes = []
        for j in range(_NCHUNK):
            gathers[j].wait()
            writes.append(
                pltpu.async_copy(
                    rows_v.at[pl.ds(j * _CHUNK, _CHUNK)],
                    out_hbm.at[pl.ds(base + j * _CHUNK, _CHUNK)],
                    sems[j],
                )
            )
        @pl.when(is_head)
        def _():
            for r in range(N_PROMPT, 24):
                for c0 in range(0, D_EMB, 16):
                    prompt_v[r, pl.ds(c0, 16)] = rows_v[r, pl.ds(c0, 16)]

        writes[0].wait()

        @pl.when(is_head)
        def _():
            prompt_cp.wait()
            pltpu.async_copy(prompt_v, out_hbm.at[pl.ds(base, 24)], psem).wait()

        @pl.when(jnp.logical_not(is_head))
        def _():
            prompt_cp.wait()

        for w in writes[1:]:
            w.wait()

    return k


_kernel_call = _make_kernel()


def kernel(x, table, prompt):
    x2 = x.reshape(_ROWS // _CHUNK, _CHUNK)
    out = _kernel_call(x2, table, prompt.reshape(N_PROMPT, D_EMB))
    return out.reshape(BATCH, SEQ_LEN, D_EMB)
